# baseline (speedup 1.0000x reference)
import jax
import jax.numpy as jnp
from jax import lax
from jax.experimental import pallas as pl
from jax.experimental.pallas import tpu as pltpu
from jax.experimental.pallas import tpu_sc as plsc

N = 10000
E = 320000
D_IN = 128
HID = 128
H1 = 8
OUT = 64
NEG = 0.2

NP = 10240
KE = 80
EP = 327680
EPT = EP // 16
CHB = 2560
NCH = EPT // CHB
BPC = CHB // KE
EPT2 = EP // 32
NCH2 = EPT2 // CHB
NPH = 10240

_MESH = dict(core_axis_name="c", subcore_axis_name="s")


def _mm1a_body(x_ref, w_ref, o_ref):
    o_ref[0] = jnp.dot(x_ref[...], w_ref[0],
                       preferred_element_type=jnp.float32,
                       precision=lax.Precision.HIGHEST)


def _mm1a(x, w1r):
    bn = 400
    return pl.pallas_call(
        _mm1a_body,
        grid=(H1, N // bn),
        in_specs=[
            pl.BlockSpec((bn, D_IN), lambda h, i: (i, 0)),
            pl.BlockSpec((1, D_IN, 128), lambda h, i: (h, 0, 0)),
        ],
        out_specs=pl.BlockSpec((1, bn, 128), lambda h, i: (h, i, 0)),
        out_shape=jax.ShapeDtypeStruct((H1, N, 128), jnp.float32),
    )(x, w1r)


def _mm1b_body(x_ref, w_ref, o_ref):
    o_ref[...] = jnp.dot(x_ref[...], w_ref[...],
                         preferred_element_type=jnp.float32,
                         precision=lax.Precision.HIGHEST)


def _mm1b(x, ulur):
    bn = 400
    return pl.pallas_call(
        _mm1b_body,
        grid=(N // bn,),
        in_specs=[
            pl.BlockSpec((bn, D_IN), lambda i: (i, 0)),
            pl.BlockSpec((D_IN, 128), lambda i: (0, 0)),
        ],
        out_specs=pl.BlockSpec((bn, 128), lambda i: (i, 0)),
        out_shape=jax.ShapeDtypeStruct((N, 128), jnp.float32),
    )(x, ulur)


def _mm2_body(h_ref, w_ref, o_ref):
    acc = jnp.zeros((400, 128), jnp.float32)
    for hh in range(H1):
        a = h_ref[hh]
        a = jnp.where(a > 0, a, jnp.exp(a) - 1.0)
        acc = acc + jnp.dot(a, w_ref[hh],
                            preferred_element_type=jnp.float32,
                            precision=lax.Precision.HIGHEST)
    o_ref[...] = acc


def _mm2(h1, w2e):
    bn = 400
    return pl.pallas_call(
        _mm2_body,
        grid=(N // bn,),
        in_specs=[
            pl.BlockSpec((H1, bn, 128), lambda i: (0, i, 0)),
            pl.BlockSpec((H1, 128, 128), lambda i: (0, 0, 0)),
        ],
        out_specs=pl.BlockSpec((bn, 128), lambda i: (i, 0)),
        out_shape=jax.ShapeDtypeStruct((N, 128), jnp.float32),
    )(h1, w2e)


def _comb_body(u_ref, s_ref, o_ref):
    bn = u_ref.shape[1]
    su = s_ref[0].reshape(bn) + s_ref[1].reshape(bn) + 1e-9
    o_ref[...] = (u_ref[0, :, :OUT] + u_ref[1, :, :OUT]) / su[:, None]


def _combine(u, s):
    bn = 1024
    return pl.pallas_call(
        _comb_body,
        grid=(NP // bn,),
        in_specs=[
            pl.BlockSpec((2, bn, 128), lambda i: (0, i, 0)),
            pl.BlockSpec((2, bn // 128, 128), lambda i: (0, i, 0)),
        ],
        out_specs=pl.BlockSpec((bn, OUT), lambda i: (i, 0)),
        out_shape=jax.ShapeDtypeStruct((NP, OUT), jnp.float32),
    )(u, s)


def _gather_dst(buf):
    return buf[7] if buf[7] is not None else buf[8]


def _stage_a(b, feat_off, el_off, er_off, src_ch, dst_ch, feat_h, elr_sh,
             buf, wait_scatter, s_sh, out_sh):
    sg, sr, dg, ds, elv, erv, wb, rbf, rf, sems = buf
    if wait_scatter is True:
        pltpu.make_async_copy(wb, s_sh.at[ds], sems[3]).wait()
        pltpu.make_async_copy(rf, out_sh.at[ds], sems[4]).wait()
    elif wait_scatter is not None:
        @pl.when(wait_scatter)
        def _():
            pltpu.make_async_copy(wb, s_sh.at[ds], sems[3]).wait()
            pltpu.make_async_copy(rf, out_sh.at[ds], sems[4]).wait()
    for v in range(KE // 16):
        sl = pl.ds(v * 16, 16)
        s16 = src_ch[pl.ds(b * KE + v * 16, 16)]
        d16 = dst_ch[pl.ds(b * KE + v * 16, 16)]
        sg[sl] = s16 + feat_off
        sr[sl] = s16 + el_off
        dg[sl] = d16 + er_off
        ds[sl] = d16
    pltpu.async_copy(feat_h.at[sg], _gather_dst(buf), sems[0])
    pltpu.async_copy(elr_sh.at[sr], elv, sems[1])
    pltpu.async_copy(elr_sh.at[dg], erv, sems[2])


def _stage_b(feat_h, elr_sh, buf, s_sh, out_sh):
    sg, sr, dg, ds, elv, erv, wb, rbf, rf, sems = buf
    pltpu.make_async_copy(elr_sh.at[sr], elv, sems[1]).wait()
    pltpu.make_async_copy(elr_sh.at[dg], erv, sems[2]).wait()
    for v in range(KE // 16):
        sl = pl.ds(v * 16, 16)
        e16 = (plsc.bitcast(elv[sl], jnp.float32)
               + plsc.bitcast(erv[sl], jnp.float32))
        wb[sl] = jnp.exp(jnp.maximum(e16, NEG * e16))
    pltpu.make_async_copy(feat_h.at[sg], _gather_dst(buf), sems[0]).wait()

    if rbf is not None:
        def scale_row(r, _):
            a = plsc.load_gather(wb, [jnp.full((16,), r, jnp.int32)])
            for q in range(4):
                ab = rbf[r, pl.ds(q * 32, 32)]
                lo, hi = plsc.unpack(ab, format=plsc.PackFormat.INTERLEAVED)
                rf[r, pl.ds(q * 32, 16)] = lo * a
                rf[r, pl.ds(q * 32 + 16, 16)] = hi * a
            return 0
    else:
        def scale_row(r, _):
            a = plsc.load_gather(wb, [jnp.full((16,), r, jnp.int32)])
            for v in range(8):
                sl = pl.ds(v * 16, 16)
                rf[r, sl] = rf[r, sl] * a
            return 0

    lax.fori_loop(0, KE, scale_row, 0)
    pltpu.async_copy(wb, s_sh.at[ds], sems[3], add=True)
    pltpu.async_copy(rf, out_sh.at[ds], sems[4], add=True)


def _drain_scatters(bufs, s_sh, out_sh):
    for buf in bufs:
        _, _, _, ds, _, _, wb, _, rf, sems = buf
        pltpu.make_async_copy(wb, s_sh.at[ds], sems[3]).wait()
        pltpu.make_async_copy(rf, out_sh.at[ds], sems[4]).wait()


def _zero_rows_buf(rw):
    def st(g, _):
        for v in range(8):
            rw[g, pl.ds(v * 16, 16)] = jnp.zeros((16,), jnp.float32)
        return 0
    lax.fori_loop(0, KE, st, 0)


def _zero_accumulators(s_id, zb, rw, s_sh, out_sh):
    for k in range(640 // KE):
        pltpu.sync_copy(zb, s_sh.at[pl.ds(s_id * 640 + k * KE, KE)])
        pltpu.sync_copy(rw, out_sh.at[pl.ds(s_id * 640 + k * KE, KE)])


def _chunk_sweep(src_h, dst_h, tile_base, nch, feat_off, el_off, er_off,
                 feat_h, elr_sh, buf0, buf1, src_ch, dst_ch, csems,
                 s_sh, out_sh):
    for ci in range(nch):
        par = ci % 2
        sc, dc = src_ch[par], dst_ch[par]
        pltpu.make_async_copy(src_h.at[pl.ds(tile_base, CHB)], sc,
                              csems[par][0]).wait()
        pltpu.make_async_copy(dst_h.at[pl.ds(tile_base, CHB)], dc,
                              csems[par][1]).wait()
        if ci + 1 < nch:
            nb = (ci + 1) % 2
            off = tile_base + (ci + 1) * CHB
            pltpu.async_copy(src_h.at[pl.ds(off, CHB)],
                             src_ch[nb], csems[nb][0])
            pltpu.async_copy(dst_h.at[pl.ds(off, CHB)],
                             dst_ch[nb], csems[nb][1])
        _stage_a(0, feat_off, el_off, er_off, sc, dc, feat_h, elr_sh, buf0,
                 None if ci == 0 else True, s_sh, out_sh)

        def pair(p, _):
            b0 = 2 * p
            _stage_a(b0 + 1, feat_off, el_off, er_off, sc, dc, feat_h,
                     elr_sh, buf1, (p > 0) if ci == 0 else True,
                     s_sh, out_sh)
            _stage_b(feat_h, elr_sh, buf0, s_sh, out_sh)

            @pl.when(p < BPC // 2 - 1)
            def _():
                _stage_a(b0 + 2, feat_off, el_off, er_off, sc, dc, feat_h,
                         elr_sh, buf0, True, s_sh, out_sh)
            _stage_b(feat_h, elr_sh, buf1, s_sh, out_sh)
            return 0

        lax.fori_loop(0, BPC // 2, pair, 0)
    _drain_scatters((buf0, buf1), s_sh, out_sh)


def _divide_chunks(s_id, out_h, out_base, s_sh, out_sh, sbuf, ibuf, rows):
    def one(half, _):
        c0 = s_id + half * 16
        pltpu.sync_copy(s_sh.at[pl.ds(c0 * 64, 64)], sbuf.at[pl.ds(0, 64)])
        pltpu.sync_copy(out_sh.at[pl.ds(c0 * 64, 64)], rows.at[pl.ds(0, 64)])
        for v in range(4):
            sl = pl.ds(v * 16, 16)
            ibuf[sl] = 1.0 / (sbuf[sl] + 1e-9)

        def scale_row(r, _):
            a = plsc.load_gather(ibuf, [jnp.full((16,), r, jnp.int32)])
            for v in range(8):
                sl = pl.ds(v * 16, 16)
                rows[r, sl] = rows[r, sl] * a
            return 0

        lax.fori_loop(0, 64, scale_row, 0)
        pltpu.sync_copy(rows.at[pl.ds(0, 64)],
                        out_h.at[pl.ds(out_base + c0 * 64, 64)])
        return 0

    lax.fori_loop(0, NP // 64 // 16, one, 0)


def _sc1_body(src_h, dst_h, elr_h, feat_h, out_h,
              src_c0, src_c1, dst_c0, dst_c1,
              sg0, sr0, dg0, ds0, el0, er0, wb0, rf0,
              sg1, sr1, dg1, ds1, el1, er1, wb1, rf1,
              elr_sh, s_sh, out_sh,
              gf0, ge0, gr0, ssw0, ssr0, gf1, ge1, gr1, ssw1, ssr1,
              cs0, cd0, cs1, cd1):
    c = lax.axis_index("c")
    s_id = lax.axis_index("s")
    buf0 = (sg0, sr0, dg0, ds0, el0, er0, wb0, None, rf0,
            (gf0, ge0, gr0, ssw0, ssr0))
    buf1 = (sg1, sr1, dg1, ds1, el1, er1, wb1, None, rf1,
            (gf1, ge1, gr1, ssw1, ssr1))
    src_ch = (src_c0, src_c1)
    dst_ch = (dst_c0, dst_c1)
    csems = ((cs0, cd0), (cs1, cd1))
    tile_base = s_id * EPT

    def head(hh, _):
        h = c * 4 + hh
        for tb in (0, NPH):
            g = h * 2 * NPH + tb + s_id * 640
            pltpu.sync_copy(elr_h.at[pl.ds(g, 640)], src_c0.at[pl.ds(0, 640)])
            pltpu.sync_copy(src_c0.at[pl.ds(0, 640)],
                            elr_sh.at[pl.ds(tb + s_id * 640, 640)])
        _zero_rows_buf(rf0)
        for v in range(KE // 16):
            wb0[pl.ds(v * 16, 16)] = jnp.zeros((16,), jnp.float32)
        _zero_accumulators(s_id, wb0, rf0, s_sh, out_sh)
        pltpu.async_copy(src_h.at[pl.ds(tile_base, CHB)], src_c0, cs0)
        pltpu.async_copy(dst_h.at[pl.ds(tile_base, CHB)], dst_c0, cd0)
        plsc.subcore_barrier()
        _chunk_sweep(src_h, dst_h, tile_base, NCH, h * N, 0, NPH,
                     feat_h, elr_sh, buf0, buf1, src_ch, dst_ch, csems,
                     s_sh, out_sh)
        plsc.subcore_barrier()
        _divide_chunks(s_id, out_h, h * NP, s_sh, out_sh, wb0, wb1, rf0)
        plsc.subcore_barrier()
        return 0

    lax.fori_loop(0, 4, head, 0)


def _sc1(src, dst, elr_i, feat):
    dma = pltpu.SemaphoreType.DMA
    ik = jnp.int32
    f = jnp.float32
    kern = pl.kernel(
        _sc1_body,
        mesh=plsc.VectorSubcoreMesh(**_MESH),
        compiler_params=pltpu.CompilerParams(needs_layout_passes=False),
        out_type=jax.ShapeDtypeStruct((H1 * NP, 128), f),
        scratch_types=[
            pltpu.VMEM((CHB,), ik), pltpu.VMEM((CHB,), ik),
            pltpu.VMEM((CHB,), ik), pltpu.VMEM((CHB,), ik),
            pltpu.VMEM((KE,), ik), pltpu.VMEM((KE,), ik),
            pltpu.VMEM((KE,), ik), pltpu.VMEM((KE,), ik),
            pltpu.VMEM((KE,), ik), pltpu.VMEM((KE,), ik),
            pltpu.VMEM((KE,), f),
            pltpu.VMEM((KE, 128), f),
            pltpu.VMEM((KE,), ik), pltpu.VMEM((KE,), ik),
            pltpu.VMEM((KE,), ik), pltpu.VMEM((KE,), ik),
            pltpu.VMEM((KE,), ik), pltpu.VMEM((KE,), ik),
            pltpu.VMEM((KE,), f),
            pltpu.VMEM((KE, 128), f),
            pltpu.VMEM_SHARED((2 * NPH,), ik),
            pltpu.VMEM_SHARED((NP,), f),
            pltpu.VMEM_SHARED((NP, 128), f),
            dma, dma, dma, dma, dma, dma, dma, dma, dma, dma,
            dma, dma, dma, dma,
        ],
    )
    return kern(src, dst, elr_i, feat)


def _sc2_body(src_h, dst_h, elr2_h, feat2_h, u_h, s_out_h,
              src_c0, src_c1, dst_c0, dst_c1,
              sg0, sr0, dg0, ds0, el0, er0, wb0, rf0,
              sg1, sr1, dg1, ds1, el1, er1, wb1, rf1,
              elr_sh, s_sh, out_sh,
              gf0, ge0, gr0, ssw0, ssr0, gf1, ge1, gr1, ssw1, ssr1,
              cs0, cd0, cs1, cd1):
    c = lax.axis_index("c")
    s_id = lax.axis_index("s")
    chunk = NP // 16
    pltpu.sync_copy(elr2_h.at[pl.ds(s_id * 1280, 1280)],
                    src_c0.at[pl.ds(0, 1280)])
    pltpu.sync_copy(src_c0.at[pl.ds(0, 1280)],
                    elr_sh.at[pl.ds(s_id * 1280, 1280)])
    _zero_rows_buf(rf0)
    for v in range(KE // 16):
        wb0[pl.ds(v * 16, 16)] = jnp.zeros((16,), jnp.float32)
    _zero_accumulators(s_id, wb0, rf0, s_sh, out_sh)
    wid = c * 16 + s_id
    tile_base = wid * EPT2
    pltpu.async_copy(src_h.at[pl.ds(tile_base, CHB)], src_c0, cs0)
    pltpu.async_copy(dst_h.at[pl.ds(tile_base, CHB)], dst_c0, cd0)
    plsc.subcore_barrier()
    buf0 = (sg0, sr0, dg0, ds0, el0, er0, wb0, None, rf0,
            (gf0, ge0, gr0, ssw0, ssr0))
    buf1 = (sg1, sr1, dg1, ds1, el1, er1, wb1, None, rf1,
            (gf1, ge1, gr1, ssw1, ssr1))
    _chunk_sweep(src_h, dst_h, tile_base, NCH2, 0, 0, N,
                 feat2_h, elr_sh, buf0, buf1, (src_c0, src_c1),
                 (dst_c0, dst_c1), ((cs0, cd0), (cs1, cd1)), s_sh, out_sh)
    plsc.subcore_barrier()
    pltpu.sync_copy(out_sh.at[pl.ds(s_id * chunk, chunk)],
                    u_h.at[pl.ds(c * NP + s_id * chunk, chunk)])
    pltpu.sync_copy(s_sh.at[pl.ds(s_id * chunk, chunk)],
                    s_out_h.at[pl.ds(c * NP + s_id * chunk, chunk)])


def _sc2(src, dst, elr2_i, feat2):
    dma = pltpu.SemaphoreType.DMA
    ik = jnp.int32
    f = jnp.float32
    kern = pl.kernel(
        _sc2_body,
        mesh=plsc.VectorSubcoreMesh(**_MESH),
        compiler_params=pltpu.CompilerParams(needs_layout_passes=False),
        out_type=[
            jax.ShapeDtypeStruct((2 * NP, 128), f),
            jax.ShapeDtypeStruct((2 * NP,), f),
        ],
        scratch_types=[
            pltpu.VMEM((CHB,), ik), pltpu.VMEM((CHB,), ik),
            pltpu.VMEM((CHB,), ik), pltpu.VMEM((CHB,), ik),
            pltpu.VMEM((KE,), ik), pltpu.VMEM((KE,), ik),
            pltpu.VMEM((KE,), ik), pltpu.VMEM((KE,), ik),
            pltpu.VMEM((KE,), ik), pltpu.VMEM((KE,), ik),
            pltpu.VMEM((KE,), f),
            pltpu.VMEM((KE, 128), f),
            pltpu.VMEM((KE,), ik), pltpu.VMEM((KE,), ik),
            pltpu.VMEM((KE,), ik), pltpu.VMEM((KE,), ik),
            pltpu.VMEM((KE,), ik), pltpu.VMEM((KE,), ik),
            pltpu.VMEM((KE,), f),
            pltpu.VMEM((KE, 128), f),
            pltpu.VMEM_SHARED((20480,), ik),
            pltpu.VMEM_SHARED((NP,), f),
            pltpu.VMEM_SHARED((NP, 128), f),
            dma, dma, dma, dma, dma, dma, dma, dma, dma, dma,
            dma, dma, dma, dma,
        ],
    )
    return kern(src, dst, elr2_i, feat2)


def kernel(x, edge_index, W1, aL1, aR1, W2, aL2, aR2):
    pad = EP - E
    src = jnp.concatenate([edge_index[0],
                           jnp.zeros((pad,), jnp.int32)])
    dst = jnp.concatenate([edge_index[1],
                           N + jnp.arange(pad, dtype=jnp.int32) % (NP - N)])

    w1r = W1.reshape(D_IN, H1, HID).transpose(1, 0, 2)
    ul1 = jnp.einsum("hdk,hk->dh", w1r, aL1)
    ur1 = jnp.einsum("hdk,hk->dh", w1r, aR1)
    ulur = jnp.concatenate([ul1, ur1, jnp.zeros((D_IN, 112), jnp.float32)], 1)

    feat = _mm1a(x, w1r)
    eler = _mm1b(x, ulur)
    feat_flat = feat.reshape(H1 * N, 128)
    ep = jnp.pad(eler[:, :16], ((0, NPH - N), (0, 0)))
    order = [0, 8, 1, 9, 2, 10, 3, 11, 4, 12, 5, 13, 6, 14, 7, 15]
    elr = ep.T[jnp.array(order)].reshape(-1)
    elr_i = lax.bitcast_convert_type(elr, jnp.int32)

    u1 = _sc1(src, dst, elr_i, feat_flat)

    w2r = W2.reshape(H1, HID, OUT)
    ul2 = (W2 @ aL2[0]).reshape(H1, HID, 1)
    ur2 = (W2 @ aR2[0]).reshape(H1, HID, 1)
    w2e = jnp.concatenate(
        [w2r, ul2, ur2, jnp.zeros((H1, HID, 62), jnp.float32)], 2)

    m2 = _mm2(u1.reshape(H1, NP, 128), w2e)
    elr2 = jnp.concatenate(
        [m2[:, 64], m2[:, 65], jnp.zeros((480,), jnp.float32)])
    elr2_i = lax.bitcast_convert_type(elr2, jnp.int32)

    u2, s2 = _sc2(src, dst, elr2_i, m2)
    logits = _combine(u2.reshape(2, NP, 128),
                      s2.reshape(2, NP // 128, 128))
    return logits[:N]


# speedup vs baseline: 1.7376x; 1.7376x over previous
import jax
import jax.numpy as jnp
from jax import lax
from jax.experimental import pallas as pl
from jax.experimental.pallas import tpu as pltpu
from jax.experimental.pallas import tpu_sc as plsc

N = 10000
E = 320000
D_IN = 128
HID = 128
H1 = 8
OUT = 64
NEG = 0.2

NP = 10240
KE = 80
EP = 327680
EPT = EP // 16
CHB = 2560
NCH = EPT // CHB
BPC = CHB // KE
EPT2 = EP // 32
NCH2 = EPT2 // CHB
NPH = 10240

_MESH = dict(core_axis_name="c", subcore_axis_name="s")


def _mm1a_body(x_ref, w_ref, o_ref):
    o_ref[0] = jnp.dot(x_ref[...], w_ref[0],
                       preferred_element_type=jnp.float32,
                       precision=lax.Precision.HIGHEST)


def _mm1a(x, w1r):
    bn = 400
    return pl.pallas_call(
        _mm1a_body,
        grid=(H1, N // bn),
        in_specs=[
            pl.BlockSpec((bn, D_IN), lambda h, i: (i, 0)),
            pl.BlockSpec((1, D_IN, 128), lambda h, i: (h, 0, 0)),
        ],
        out_specs=pl.BlockSpec((1, bn, 128), lambda h, i: (h, i, 0)),
        out_shape=jax.ShapeDtypeStruct((H1, N, 128), jnp.float32),
    )(x, w1r)


def _mm1b_body(x_ref, w_ref, o_ref):
    o_ref[...] = jnp.dot(x_ref[...], w_ref[...],
                         preferred_element_type=jnp.float32,
                         precision=lax.Precision.HIGHEST)


def _mm1b(x, ulur):
    bn = 400
    return pl.pallas_call(
        _mm1b_body,
        grid=(N // bn,),
        in_specs=[
            pl.BlockSpec((bn, D_IN), lambda i: (i, 0)),
            pl.BlockSpec((D_IN, 128), lambda i: (0, 0)),
        ],
        out_specs=pl.BlockSpec((bn, 128), lambda i: (i, 0)),
        out_shape=jax.ShapeDtypeStruct((N, 128), jnp.float32),
    )(x, ulur)


def _mm2_body(h_ref, w_ref, o_ref):
    acc = jnp.zeros((400, 128), jnp.float32)
    for hh in range(H1):
        a = h_ref[hh]
        a = jnp.where(a > 0, a, jnp.exp(a) - 1.0)
        acc = acc + jnp.dot(a, w_ref[hh],
                            preferred_element_type=jnp.float32,
                            precision=lax.Precision.HIGHEST)
    o_ref[...] = acc


def _mm2(h1, w2e):
    bn = 400
    return pl.pallas_call(
        _mm2_body,
        grid=(N // bn,),
        in_specs=[
            pl.BlockSpec((H1, bn, 128), lambda i: (0, i, 0)),
            pl.BlockSpec((H1, 128, 128), lambda i: (0, 0, 0)),
        ],
        out_specs=pl.BlockSpec((bn, 128), lambda i: (i, 0)),
        out_shape=jax.ShapeDtypeStruct((N, 128), jnp.float32),
    )(h1, w2e)


def _comb_body(u_ref, s_ref, o_ref):
    bn = u_ref.shape[1]
    su = s_ref[0].reshape(bn) + s_ref[1].reshape(bn) + 1e-9
    o_ref[...] = (u_ref[0, :, :OUT] + u_ref[1, :, :OUT]) / su[:, None]


def _combine(u, s):
    bn = 1024
    return pl.pallas_call(
        _comb_body,
        grid=(NP // bn,),
        in_specs=[
            pl.BlockSpec((2, bn, 128), lambda i: (0, i, 0)),
            pl.BlockSpec((2, bn // 128, 128), lambda i: (0, i, 0)),
        ],
        out_specs=pl.BlockSpec((bn, OUT), lambda i: (i, 0)),
        out_shape=jax.ShapeDtypeStruct((NP, OUT), jnp.float32),
    )(u, s)


def _gather_dst(buf):
    return buf[7] if buf[7] is not None else buf[8]


def _stage_a(b, feat_off, el_off, er_off, src_ch, dst_ch, feat_h, elr_sh,
             buf, wait_scatter, s_sh, out_sh):
    sg, sr, dg, ds, elv, erv, wb, rbf, rf, sems = buf
    if wait_scatter is True:
        pltpu.make_async_copy(wb, s_sh.at[ds], sems[3]).wait()
        pltpu.make_async_copy(rf, out_sh.at[ds], sems[4]).wait()
    elif wait_scatter is not None:
        @pl.when(wait_scatter)
        def _():
            pltpu.make_async_copy(wb, s_sh.at[ds], sems[3]).wait()
            pltpu.make_async_copy(rf, out_sh.at[ds], sems[4]).wait()
    for v in range(KE // 16):
        sl = pl.ds(v * 16, 16)
        s16 = src_ch[pl.ds(b * KE + v * 16, 16)]
        d16 = dst_ch[pl.ds(b * KE + v * 16, 16)]
        sg[sl] = s16 + feat_off
        sr[sl] = s16 + el_off
        dg[sl] = d16 + er_off
        ds[sl] = d16
    pltpu.async_copy(feat_h.at[sg], _gather_dst(buf), sems[0])
    pltpu.async_copy(elr_sh.at[sr], elv, sems[1])
    pltpu.async_copy(elr_sh.at[dg], erv, sems[2])


def _stage_b(feat_h, elr_sh, buf, s_sh, out_sh):
    sg, sr, dg, ds, elv, erv, wb, rbf, rf, sems = buf
    pltpu.make_async_copy(elr_sh.at[sr], elv, sems[1]).wait()
    pltpu.make_async_copy(elr_sh.at[dg], erv, sems[2]).wait()
    for v in range(KE // 16):
        sl = pl.ds(v * 16, 16)
        e16 = (plsc.bitcast(elv[sl], jnp.float32)
               + plsc.bitcast(erv[sl], jnp.float32))
        wb[sl] = jnp.exp(jnp.maximum(e16, NEG * e16))
    pltpu.make_async_copy(feat_h.at[sg], _gather_dst(buf), sems[0]).wait()

    if rbf is not None:
        def scale_row(r, _):
            a = plsc.load_gather(wb, [jnp.full((16,), r, jnp.int32)])
            for q in range(4):
                ab = rbf[r, pl.ds(q * 32, 32)]
                lo, hi = plsc.unpack(ab, format=plsc.PackFormat.INTERLEAVED)
                rf[r, pl.ds(q * 32, 16)] = lo * a
                rf[r, pl.ds(q * 32 + 16, 16)] = hi * a
            return 0
    else:
        def scale_row(r, _):
            a = plsc.load_gather(wb, [jnp.full((16,), r, jnp.int32)])
            for v in range(8):
                sl = pl.ds(v * 16, 16)
                rf[r, sl] = rf[r, sl] * a
            return 0

    lax.fori_loop(0, KE, scale_row, 0)
    pltpu.async_copy(wb, s_sh.at[ds], sems[3], add=True)
    pltpu.async_copy(rf, out_sh.at[ds], sems[4], add=True)


def _drain_scatters(bufs, s_sh, out_sh):
    for buf in bufs:
        _, _, _, ds, _, _, wb, _, rf, sems = buf
        pltpu.make_async_copy(wb, s_sh.at[ds], sems[3]).wait()
        pltpu.make_async_copy(rf, out_sh.at[ds], sems[4]).wait()


def _zero_rows_buf(rw):
    def st(g, _):
        for v in range(8):
            rw[g, pl.ds(v * 16, 16)] = jnp.zeros((16,), jnp.float32)
        return 0
    lax.fori_loop(0, KE, st, 0)


def _zero_accumulators(s_id, zb, rw, s_sh, out_sh):
    for k in range(640 // KE):
        pltpu.sync_copy(zb, s_sh.at[pl.ds(s_id * 640 + k * KE, KE)])
        pltpu.sync_copy(rw, out_sh.at[pl.ds(s_id * 640 + k * KE, KE)])


def _chunk_sweep(src_h, dst_h, tile_base, nch, feat_off, el_off, er_off,
                 feat_h, elr_sh, buf0, buf1, src_ch, dst_ch, csems,
                 s_sh, out_sh):
    for ci in range(nch):
        par = ci % 2
        sc, dc = src_ch[par], dst_ch[par]
        pltpu.make_async_copy(src_h.at[pl.ds(tile_base, CHB)], sc,
                              csems[par][0]).wait()
        pltpu.make_async_copy(dst_h.at[pl.ds(tile_base, CHB)], dc,
                              csems[par][1]).wait()
        if ci + 1 < nch:
            nb = (ci + 1) % 2
            off = tile_base + (ci + 1) * CHB
            pltpu.async_copy(src_h.at[pl.ds(off, CHB)],
                             src_ch[nb], csems[nb][0])
            pltpu.async_copy(dst_h.at[pl.ds(off, CHB)],
                             dst_ch[nb], csems[nb][1])
        _stage_a(0, feat_off, el_off, er_off, sc, dc, feat_h, elr_sh, buf0,
                 None if ci == 0 else True, s_sh, out_sh)

        def pair(p, _):
            b0 = 2 * p
            _stage_a(b0 + 1, feat_off, el_off, er_off, sc, dc, feat_h,
                     elr_sh, buf1, (p > 0) if ci == 0 else True,
                     s_sh, out_sh)
            _stage_b(feat_h, elr_sh, buf0, s_sh, out_sh)

            @pl.when(p < BPC // 2 - 1)
            def _():
                _stage_a(b0 + 2, feat_off, el_off, er_off, sc, dc, feat_h,
                         elr_sh, buf0, True, s_sh, out_sh)
            _stage_b(feat_h, elr_sh, buf1, s_sh, out_sh)
            return 0

        lax.fori_loop(0, BPC // 2, pair, 0)
    _drain_scatters((buf0, buf1), s_sh, out_sh)


def _divide_chunks(s_id, out_h, out_base, s_sh, out_sh, sbuf, ibuf, rows):
    def one(half, _):
        c0 = s_id + half * 16
        pltpu.sync_copy(s_sh.at[pl.ds(c0 * 64, 64)], sbuf.at[pl.ds(0, 64)])
        pltpu.sync_copy(out_sh.at[pl.ds(c0 * 64, 64)], rows.at[pl.ds(0, 64)])
        for v in range(4):
            sl = pl.ds(v * 16, 16)
            ibuf[sl] = 1.0 / (sbuf[sl] + 1e-9)

        def scale_row(r, _):
            a = plsc.load_gather(ibuf, [jnp.full((16,), r, jnp.int32)])
            for v in range(8):
                sl = pl.ds(v * 16, 16)
                rows[r, sl] = rows[r, sl] * a
            return 0

        lax.fori_loop(0, 64, scale_row, 0)
        pltpu.sync_copy(rows.at[pl.ds(0, 64)],
                        out_h.at[pl.ds(out_base + c0 * 64, 64)])
        return 0

    lax.fori_loop(0, NP // 64 // 16, one, 0)


def _sc1_body(src_h, dst_h, elr_h, feat_h, out_h,
              src_c0, src_c1, dst_c0, dst_c1,
              sg0, sr0, dg0, ds0, el0, er0, wb0, rf0,
              sg1, sr1, dg1, ds1, el1, er1, wb1, rf1,
              elr_sh, s_sh, out_sh,
              gf0, ge0, gr0, ssw0, ssr0, gf1, ge1, gr1, ssw1, ssr1,
              cs0, cd0, cs1, cd1):
    c = lax.axis_index("c")
    s_id = lax.axis_index("s")
    buf0 = (sg0, sr0, dg0, ds0, el0, er0, wb0, None, rf0,
            (gf0, ge0, gr0, ssw0, ssr0))
    buf1 = (sg1, sr1, dg1, ds1, el1, er1, wb1, None, rf1,
            (gf1, ge1, gr1, ssw1, ssr1))
    src_ch = (src_c0, src_c1)
    dst_ch = (dst_c0, dst_c1)
    csems = ((cs0, cd0), (cs1, cd1))
    tile_base = s_id * EPT

    def head(hh, _):
        h = c * 4 + hh
        for tb in (0, NPH):
            g = h * 2 * NPH + tb + s_id * 640
            pltpu.sync_copy(elr_h.at[pl.ds(g, 640)], src_c0.at[pl.ds(0, 640)])
            pltpu.sync_copy(src_c0.at[pl.ds(0, 640)],
                            elr_sh.at[pl.ds(tb + s_id * 640, 640)])
        _zero_rows_buf(rf0)
        for v in range(KE // 16):
            wb0[pl.ds(v * 16, 16)] = jnp.zeros((16,), jnp.float32)
        _zero_accumulators(s_id, wb0, rf0, s_sh, out_sh)
        pltpu.async_copy(src_h.at[pl.ds(tile_base, CHB)], src_c0, cs0)
        pltpu.async_copy(dst_h.at[pl.ds(tile_base, CHB)], dst_c0, cd0)
        plsc.subcore_barrier()
        _chunk_sweep(src_h, dst_h, tile_base, NCH, h * N, 0, NPH,
                     feat_h, elr_sh, buf0, buf1, src_ch, dst_ch, csems,
                     s_sh, out_sh)
        plsc.subcore_barrier()
        _divide_chunks(s_id, out_h, h * NP, s_sh, out_sh, wb0, wb1, rf0)
        plsc.subcore_barrier()
        return 0

    lax.fori_loop(0, 4, head, 0)


def _sc1(src, dst, elr_i, feat):
    dma = pltpu.SemaphoreType.DMA
    ik = jnp.int32
    f = jnp.float32
    kern = pl.kernel(
        _sc1_body,
        mesh=plsc.VectorSubcoreMesh(**_MESH),
        compiler_params=pltpu.CompilerParams(needs_layout_passes=False),
        out_type=jax.ShapeDtypeStruct((H1 * NP, 128), f),
        scratch_types=[
            pltpu.VMEM((CHB,), ik), pltpu.VMEM((CHB,), ik),
            pltpu.VMEM((CHB,), ik), pltpu.VMEM((CHB,), ik),
            pltpu.VMEM((KE,), ik), pltpu.VMEM((KE,), ik),
            pltpu.VMEM((KE,), ik), pltpu.VMEM((KE,), ik),
            pltpu.VMEM((KE,), ik), pltpu.VMEM((KE,), ik),
            pltpu.VMEM((KE,), f),
            pltpu.VMEM((KE, 128), f),
            pltpu.VMEM((KE,), ik), pltpu.VMEM((KE,), ik),
            pltpu.VMEM((KE,), ik), pltpu.VMEM((KE,), ik),
            pltpu.VMEM((KE,), ik), pltpu.VMEM((KE,), ik),
            pltpu.VMEM((KE,), f),
            pltpu.VMEM((KE, 128), f),
            pltpu.VMEM_SHARED((2 * NPH,), ik),
            pltpu.VMEM_SHARED((NP,), f),
            pltpu.VMEM_SHARED((NP, 128), f),
            dma, dma, dma, dma, dma, dma, dma, dma, dma, dma,
            dma, dma, dma, dma,
        ],
    )
    return kern(src, dst, elr_i, feat)


def _sc2_body(src_h, dst_h, elr2_h, feat2_h, u_h, s_out_h,
              src_c0, src_c1, dst_c0, dst_c1,
              sg0, sr0, dg0, ds0, el0, er0, wb0, rf0,
              sg1, sr1, dg1, ds1, el1, er1, wb1, rf1,
              elr_sh, s_sh, out_sh,
              gf0, ge0, gr0, ssw0, ssr0, gf1, ge1, gr1, ssw1, ssr1,
              cs0, cd0, cs1, cd1):
    c = lax.axis_index("c")
    s_id = lax.axis_index("s")
    chunk = NP // 16
    pltpu.sync_copy(elr2_h.at[pl.ds(s_id * 1280, 1280)],
                    src_c0.at[pl.ds(0, 1280)])
    pltpu.sync_copy(src_c0.at[pl.ds(0, 1280)],
                    elr_sh.at[pl.ds(s_id * 1280, 1280)])
    _zero_rows_buf(rf0)
    for v in range(KE // 16):
        wb0[pl.ds(v * 16, 16)] = jnp.zeros((16,), jnp.float32)
    _zero_accumulators(s_id, wb0, rf0, s_sh, out_sh)
    wid = c * 16 + s_id
    tile_base = wid * EPT2
    pltpu.async_copy(src_h.at[pl.ds(tile_base, CHB)], src_c0, cs0)
    pltpu.async_copy(dst_h.at[pl.ds(tile_base, CHB)], dst_c0, cd0)
    plsc.subcore_barrier()
    buf0 = (sg0, sr0, dg0, ds0, el0, er0, wb0, None, rf0,
            (gf0, ge0, gr0, ssw0, ssr0))
    buf1 = (sg1, sr1, dg1, ds1, el1, er1, wb1, None, rf1,
            (gf1, ge1, gr1, ssw1, ssr1))
    _chunk_sweep(src_h, dst_h, tile_base, NCH2, 0, 0, N,
                 feat2_h, elr_sh, buf0, buf1, (src_c0, src_c1),
                 (dst_c0, dst_c1), ((cs0, cd0), (cs1, cd1)), s_sh, out_sh)
    plsc.subcore_barrier()
    pltpu.sync_copy(out_sh.at[pl.ds(s_id * chunk, chunk)],
                    u_h.at[pl.ds(c * NP + s_id * chunk, chunk)])
    pltpu.sync_copy(s_sh.at[pl.ds(s_id * chunk, chunk)],
                    s_out_h.at[pl.ds(c * NP + s_id * chunk, chunk)])


def _sc2(src, dst, elr2_i, feat2):
    dma = pltpu.SemaphoreType.DMA
    ik = jnp.int32
    f = jnp.float32
    kern = pl.kernel(
        _sc2_body,
        mesh=plsc.VectorSubcoreMesh(**_MESH),
        compiler_params=pltpu.CompilerParams(needs_layout_passes=False),
        out_type=[
            jax.ShapeDtypeStruct((2 * NP, 128), f),
            jax.ShapeDtypeStruct((2 * NP,), f),
        ],
        scratch_types=[
            pltpu.VMEM((CHB,), ik), pltpu.VMEM((CHB,), ik),
            pltpu.VMEM((CHB,), ik), pltpu.VMEM((CHB,), ik),
            pltpu.VMEM((KE,), ik), pltpu.VMEM((KE,), ik),
            pltpu.VMEM((KE,), ik), pltpu.VMEM((KE,), ik),
            pltpu.VMEM((KE,), ik), pltpu.VMEM((KE,), ik),
            pltpu.VMEM((KE,), f),
            pltpu.VMEM((KE, 128), f),
            pltpu.VMEM((KE,), ik), pltpu.VMEM((KE,), ik),
            pltpu.VMEM((KE,), ik), pltpu.VMEM((KE,), ik),
            pltpu.VMEM((KE,), ik), pltpu.VMEM((KE,), ik),
            pltpu.VMEM((KE,), f),
            pltpu.VMEM((KE, 128), f),
            pltpu.VMEM_SHARED((20480,), ik),
            pltpu.VMEM_SHARED((NP,), f),
            pltpu.VMEM_SHARED((NP, 128), f),
            dma, dma, dma, dma, dma, dma, dma, dma, dma, dma,
            dma, dma, dma, dma,
        ],
    )
    return kern(src, dst, elr2_i, feat2)


def kernel(x, edge_index, W1, aL1, aR1, W2, aL2, aR2):
    pad = EP - E
    pidx = jnp.arange(pad, dtype=jnp.int32)
    src = jnp.concatenate([edge_index[0], pidx * 37 % N])
    dst = jnp.concatenate([edge_index[1], N + pidx % (NP - N)])

    w1r = W1.reshape(D_IN, H1, HID).transpose(1, 0, 2)
    ul1 = jnp.einsum("hdk,hk->dh", w1r, aL1)
    ur1 = jnp.einsum("hdk,hk->dh", w1r, aR1)
    ulur = jnp.concatenate([ul1, ur1, jnp.zeros((D_IN, 112), jnp.float32)], 1)

    feat = _mm1a(x, w1r)
    eler = _mm1b(x, ulur)
    feat_flat = feat.reshape(H1 * N, 128)
    ep = jnp.pad(eler[:, :16], ((0, NPH - N), (0, 0)))
    order = [0, 8, 1, 9, 2, 10, 3, 11, 4, 12, 5, 13, 6, 14, 7, 15]
    elr = ep.T[jnp.array(order)].reshape(-1)
    elr_i = lax.bitcast_convert_type(elr, jnp.int32)

    u1 = _sc1(src, dst, elr_i, feat_flat)

    w2r = W2.reshape(H1, HID, OUT)
    ul2 = (W2 @ aL2[0]).reshape(H1, HID, 1)
    ur2 = (W2 @ aR2[0]).reshape(H1, HID, 1)
    w2e = jnp.concatenate(
        [w2r, ul2, ur2, jnp.zeros((H1, HID, 62), jnp.float32)], 2)

    m2 = _mm2(u1.reshape(H1, NP, 128), w2e)
    elr2 = jnp.concatenate(
        [m2[:, 64], m2[:, 65], jnp.zeros((480,), jnp.float32)])
    elr2_i = lax.bitcast_convert_type(elr2, jnp.int32)

    u2, s2 = _sc2(src, dst, elr2_i, m2)
    logits = _combine(u2.reshape(2, NP, 128),
                      s2.reshape(2, NP // 128, 128))
    return logits[:N]


# speedup vs baseline: 1.7799x; 1.0244x over previous
import jax
import jax.numpy as jnp
from jax import lax
from jax.experimental import pallas as pl
from jax.experimental.pallas import tpu as pltpu
from jax.experimental.pallas import tpu_sc as plsc

N = 10000
E = 320000
D_IN = 128
HID = 128
H1 = 8
OUT = 64
NEG = 0.2

NP = 10240
KE = 128
EP = 327680
EPT = EP // 16
CHB = 2560
NCH = EPT // CHB
BPC = CHB // KE
EPT2 = EP // 32
NCH2 = EPT2 // CHB
NPH = 10240

_MESH = dict(core_axis_name="c", subcore_axis_name="s")


def _mm1a_body(x_ref, w_ref, o_ref):
    o_ref[0] = jnp.dot(x_ref[...], w_ref[0],
                       preferred_element_type=jnp.float32,
                       precision=lax.Precision.HIGHEST)


def _mm1a(x, w1r):
    bn = 400
    return pl.pallas_call(
        _mm1a_body,
        grid=(H1, N // bn),
        in_specs=[
            pl.BlockSpec((bn, D_IN), lambda h, i: (i, 0)),
            pl.BlockSpec((1, D_IN, 128), lambda h, i: (h, 0, 0)),
        ],
        out_specs=pl.BlockSpec((1, bn, 128), lambda h, i: (h, i, 0)),
        out_shape=jax.ShapeDtypeStruct((H1, N, 128), jnp.float32),
    )(x, w1r)


def _mm1b_body(x_ref, w_ref, o_ref):
    o_ref[...] = jnp.dot(x_ref[...], w_ref[...],
                         preferred_element_type=jnp.float32,
                         precision=lax.Precision.HIGHEST)


def _mm1b(x, ulur):
    bn = 400
    return pl.pallas_call(
        _mm1b_body,
        grid=(N // bn,),
        in_specs=[
            pl.BlockSpec((bn, D_IN), lambda i: (i, 0)),
            pl.BlockSpec((D_IN, 128), lambda i: (0, 0)),
        ],
        out_specs=pl.BlockSpec((bn, 128), lambda i: (i, 0)),
        out_shape=jax.ShapeDtypeStruct((N, 128), jnp.float32),
    )(x, ulur)


def _mm2_body(h_ref, w_ref, o_ref):
    acc = jnp.zeros((400, 128), jnp.float32)
    for hh in range(H1):
        a = h_ref[hh]
        a = jnp.where(a > 0, a, jnp.exp(a) - 1.0)
        acc = acc + jnp.dot(a, w_ref[hh],
                            preferred_element_type=jnp.float32,
                            precision=lax.Precision.HIGHEST)
    o_ref[...] = acc


def _mm2(h1, w2e):
    bn = 400
    return pl.pallas_call(
        _mm2_body,
        grid=(N // bn,),
        in_specs=[
            pl.BlockSpec((H1, bn, 128), lambda i: (0, i, 0)),
            pl.BlockSpec((H1, 128, 128), lambda i: (0, 0, 0)),
        ],
        out_specs=pl.BlockSpec((bn, 128), lambda i: (i, 0)),
        out_shape=jax.ShapeDtypeStruct((N, 128), jnp.float32),
    )(h1, w2e)


def _comb_body(u_ref, s_ref, o_ref):
    bn = u_ref.shape[1]
    su = s_ref[0].reshape(bn) + s_ref[1].reshape(bn) + 1e-9
    o_ref[...] = (u_ref[0, :, :OUT] + u_ref[1, :, :OUT]) / su[:, None]


def _combine(u, s):
    bn = 1024
    return pl.pallas_call(
        _comb_body,
        grid=(NP // bn,),
        in_specs=[
            pl.BlockSpec((2, bn, 128), lambda i: (0, i, 0)),
            pl.BlockSpec((2, bn // 128, 128), lambda i: (0, i, 0)),
        ],
        out_specs=pl.BlockSpec((bn, OUT), lambda i: (i, 0)),
        out_shape=jax.ShapeDtypeStruct((NP, OUT), jnp.float32),
    )(u, s)


def _gather_dst(buf):
    return buf[7] if buf[7] is not None else buf[8]


def _stage_a(b, feat_off, el_off, er_off, src_ch, dst_ch, feat_h, elr_sh,
             buf, wait_scatter, s_sh, out_sh):
    sg, sr, dg, ds, elv, erv, wb, rbf, rf, sems = buf
    if wait_scatter is True:
        pltpu.make_async_copy(wb, s_sh.at[ds], sems[3]).wait()
        pltpu.make_async_copy(rf, out_sh.at[ds], sems[4]).wait()
    elif wait_scatter is not None:
        @pl.when(wait_scatter)
        def _():
            pltpu.make_async_copy(wb, s_sh.at[ds], sems[3]).wait()
            pltpu.make_async_copy(rf, out_sh.at[ds], sems[4]).wait()
    for v in range(KE // 16):
        sl = pl.ds(v * 16, 16)
        s16 = src_ch[pl.ds(b * KE + v * 16, 16)]
        d16 = dst_ch[pl.ds(b * KE + v * 16, 16)]
        sg[sl] = s16 + feat_off
        sr[sl] = s16 + el_off
        dg[sl] = d16 + er_off
        ds[sl] = d16
    pltpu.async_copy(feat_h.at[sg], _gather_dst(buf), sems[0])
    pltpu.async_copy(elr_sh.at[sr], elv, sems[1])
    pltpu.async_copy(elr_sh.at[dg], erv, sems[2])


def _stage_b(feat_h, elr_sh, buf, s_sh, out_sh):
    sg, sr, dg, ds, elv, erv, wb, rbf, rf, sems = buf
    pltpu.make_async_copy(elr_sh.at[sr], elv, sems[1]).wait()
    pltpu.make_async_copy(elr_sh.at[dg], erv, sems[2]).wait()
    for v in range(KE // 16):
        sl = pl.ds(v * 16, 16)
        e16 = (plsc.bitcast(elv[sl], jnp.float32)
               + plsc.bitcast(erv[sl], jnp.float32))
        wb[sl] = jnp.exp(jnp.maximum(e16, NEG * e16))
    pltpu.make_async_copy(feat_h.at[sg], _gather_dst(buf), sems[0]).wait()

    if rbf is not None:
        def scale_row(r, _):
            a = plsc.load_gather(wb, [jnp.full((16,), r, jnp.int32)])
            for q in range(4):
                ab = rbf[r, pl.ds(q * 32, 32)]
                lo, hi = plsc.unpack(ab, format=plsc.PackFormat.INTERLEAVED)
                rf[r, pl.ds(q * 32, 16)] = lo * a
                rf[r, pl.ds(q * 32 + 16, 16)] = hi * a
            return 0
    else:
        def scale_row(r, _):
            a = plsc.load_gather(wb, [jnp.full((16,), r, jnp.int32)])
            for v in range(8):
                sl = pl.ds(v * 16, 16)
                rf[r, sl] = rf[r, sl] * a
            return 0

    lax.fori_loop(0, KE, scale_row, 0)
    pltpu.async_copy(wb, s_sh.at[ds], sems[3], add=True)
    pltpu.async_copy(rf, out_sh.at[ds], sems[4], add=True)


def _drain_scatters(bufs, s_sh, out_sh):
    for buf in bufs:
        _, _, _, ds, _, _, wb, _, rf, sems = buf
        pltpu.make_async_copy(wb, s_sh.at[ds], sems[3]).wait()
        pltpu.make_async_copy(rf, out_sh.at[ds], sems[4]).wait()


def _zero_rows_buf(rw):
    def st(g, _):
        for v in range(8):
            rw[g, pl.ds(v * 16, 16)] = jnp.zeros((16,), jnp.float32)
        return 0
    lax.fori_loop(0, KE, st, 0)


def _zero_accumulators(s_id, zb, rw, s_sh, out_sh):
    for k in range(640 // KE):
        pltpu.sync_copy(zb, s_sh.at[pl.ds(s_id * 640 + k * KE, KE)])
        pltpu.sync_copy(rw, out_sh.at[pl.ds(s_id * 640 + k * KE, KE)])


def _chunk_sweep(src_h, dst_h, tile_base, nch, feat_off, el_off, er_off,
                 feat_h, elr_sh, buf0, buf1, src_ch, dst_ch, csems,
                 s_sh, out_sh):
    for ci in range(nch):
        par = ci % 2
        sc, dc = src_ch[par], dst_ch[par]
        pltpu.make_async_copy(src_h.at[pl.ds(tile_base, CHB)], sc,
                              csems[par][0]).wait()
        pltpu.make_async_copy(dst_h.at[pl.ds(tile_base, CHB)], dc,
                              csems[par][1]).wait()
        if ci + 1 < nch:
            nb = (ci + 1) % 2
            off = tile_base + (ci + 1) * CHB
            pltpu.async_copy(src_h.at[pl.ds(off, CHB)],
                             src_ch[nb], csems[nb][0])
            pltpu.async_copy(dst_h.at[pl.ds(off, CHB)],
                             dst_ch[nb], csems[nb][1])
        _stage_a(0, feat_off, el_off, er_off, sc, dc, feat_h, elr_sh, buf0,
                 None if ci == 0 else True, s_sh, out_sh)

        def pair(p, _):
            b0 = 2 * p
            _stage_a(b0 + 1, feat_off, el_off, er_off, sc, dc, feat_h,
                     elr_sh, buf1, (p > 0) if ci == 0 else True,
                     s_sh, out_sh)
            _stage_b(feat_h, elr_sh, buf0, s_sh, out_sh)

            @pl.when(p < BPC // 2 - 1)
            def _():
                _stage_a(b0 + 2, feat_off, el_off, er_off, sc, dc, feat_h,
                         elr_sh, buf0, True, s_sh, out_sh)
            _stage_b(feat_h, elr_sh, buf1, s_sh, out_sh)
            return 0

        lax.fori_loop(0, BPC // 2, pair, 0)
    _drain_scatters((buf0, buf1), s_sh, out_sh)


def _divide_chunks(s_id, out_h, out_base, s_sh, out_sh, sbuf, ibuf, rows):
    def one(half, _):
        c0 = s_id + half * 16
        pltpu.sync_copy(s_sh.at[pl.ds(c0 * 64, 64)], sbuf.at[pl.ds(0, 64)])
        pltpu.sync_copy(out_sh.at[pl.ds(c0 * 64, 64)], rows.at[pl.ds(0, 64)])
        for v in range(4):
            sl = pl.ds(v * 16, 16)
            ibuf[sl] = 1.0 / (sbuf[sl] + 1e-9)

        def scale_row(r, _):
            a = plsc.load_gather(ibuf, [jnp.full((16,), r, jnp.int32)])
            for v in range(8):
                sl = pl.ds(v * 16, 16)
                rows[r, sl] = rows[r, sl] * a
            return 0

        lax.fori_loop(0, 64, scale_row, 0)
        pltpu.sync_copy(rows.at[pl.ds(0, 64)],
                        out_h.at[pl.ds(out_base + c0 * 64, 64)])
        return 0

    lax.fori_loop(0, NP // 64 // 16, one, 0)


def _sc1_body(src_h, dst_h, elr_h, feat_h, out_h,
              src_c0, src_c1, dst_c0, dst_c1,
              sg0, sr0, dg0, ds0, el0, er0, wb0, rf0,
              sg1, sr1, dg1, ds1, el1, er1, wb1, rf1,
              elr_sh, s_sh, out_sh,
              gf0, ge0, gr0, ssw0, ssr0, gf1, ge1, gr1, ssw1, ssr1,
              cs0, cd0, cs1, cd1):
    c = lax.axis_index("c")
    s_id = lax.axis_index("s")
    buf0 = (sg0, sr0, dg0, ds0, el0, er0, wb0, None, rf0,
            (gf0, ge0, gr0, ssw0, ssr0))
    buf1 = (sg1, sr1, dg1, ds1, el1, er1, wb1, None, rf1,
            (gf1, ge1, gr1, ssw1, ssr1))
    src_ch = (src_c0, src_c1)
    dst_ch = (dst_c0, dst_c1)
    csems = ((cs0, cd0), (cs1, cd1))
    tile_base = s_id * EPT

    def head(hh, _):
        h = c * 4 + hh
        for tb in (0, NPH):
            g = h * 2 * NPH + tb + s_id * 640
            pltpu.sync_copy(elr_h.at[pl.ds(g, 640)], src_c0.at[pl.ds(0, 640)])
            pltpu.sync_copy(src_c0.at[pl.ds(0, 640)],
                            elr_sh.at[pl.ds(tb + s_id * 640, 640)])
        _zero_rows_buf(rf0)
        for v in range(KE // 16):
            wb0[pl.ds(v * 16, 16)] = jnp.zeros((16,), jnp.float32)
        _zero_accumulators(s_id, wb0, rf0, s_sh, out_sh)
        pltpu.async_copy(src_h.at[pl.ds(tile_base, CHB)], src_c0, cs0)
        pltpu.async_copy(dst_h.at[pl.ds(tile_base, CHB)], dst_c0, cd0)
        plsc.subcore_barrier()
        _chunk_sweep(src_h, dst_h, tile_base, NCH, h * N, 0, NPH,
                     feat_h, elr_sh, buf0, buf1, src_ch, dst_ch, csems,
                     s_sh, out_sh)
        plsc.subcore_barrier()
        _divide_chunks(s_id, out_h, h * NP, s_sh, out_sh, wb0, wb1, rf0)
        plsc.subcore_barrier()
        return 0

    lax.fori_loop(0, 4, head, 0)


def _sc1(src, dst, elr_i, feat):
    dma = pltpu.SemaphoreType.DMA
    ik = jnp.int32
    f = jnp.float32
    kern = pl.kernel(
        _sc1_body,
        mesh=plsc.VectorSubcoreMesh(**_MESH),
        compiler_params=pltpu.CompilerParams(needs_layout_passes=False),
        out_type=jax.ShapeDtypeStruct((H1 * NP, 128), f),
        scratch_types=[
            pltpu.VMEM((CHB,), ik), pltpu.VMEM((CHB,), ik),
            pltpu.VMEM((CHB,), ik), pltpu.VMEM((CHB,), ik),
            pltpu.VMEM((KE,), ik), pltpu.VMEM((KE,), ik),
            pltpu.VMEM((KE,), ik), pltpu.VMEM((KE,), ik),
            pltpu.VMEM((KE,), ik), pltpu.VMEM((KE,), ik),
            pltpu.VMEM((KE,), f),
            pltpu.VMEM((KE, 128), f),
            pltpu.VMEM((KE,), ik), pltpu.VMEM((KE,), ik),
            pltpu.VMEM((KE,), ik), pltpu.VMEM((KE,), ik),
            pltpu.VMEM((KE,), ik), pltpu.VMEM((KE,), ik),
            pltpu.VMEM((KE,), f),
            pltpu.VMEM((KE, 128), f),
            pltpu.VMEM_SHARED((2 * NPH,), ik),
            pltpu.VMEM_SHARED((NP,), f),
            pltpu.VMEM_SHARED((NP, 128), f),
            dma, dma, dma, dma, dma, dma, dma, dma, dma, dma,
            dma, dma, dma, dma,
        ],
    )
    return kern(src, dst, elr_i, feat)


def _sc2_body(src_h, dst_h, elr2_h, feat2_h, u_h, s_out_h,
              src_c0, src_c1, dst_c0, dst_c1,
              sg0, sr0, dg0, ds0, el0, er0, wb0, rf0,
              sg1, sr1, dg1, ds1, el1, er1, wb1, rf1,
              elr_sh, s_sh, out_sh,
              gf0, ge0, gr0, ssw0, ssr0, gf1, ge1, gr1, ssw1, ssr1,
              cs0, cd0, cs1, cd1):
    c = lax.axis_index("c")
    s_id = lax.axis_index("s")
    chunk = NP // 16
    pltpu.sync_copy(elr2_h.at[pl.ds(s_id * 1280, 1280)],
                    src_c0.at[pl.ds(0, 1280)])
    pltpu.sync_copy(src_c0.at[pl.ds(0, 1280)],
                    elr_sh.at[pl.ds(s_id * 1280, 1280)])
    _zero_rows_buf(rf0)
    for v in range(KE // 16):
        wb0[pl.ds(v * 16, 16)] = jnp.zeros((16,), jnp.float32)
    _zero_accumulators(s_id, wb0, rf0, s_sh, out_sh)
    wid = c * 16 + s_id
    tile_base = wid * EPT2
    pltpu.async_copy(src_h.at[pl.ds(tile_base, CHB)], src_c0, cs0)
    pltpu.async_copy(dst_h.at[pl.ds(tile_base, CHB)], dst_c0, cd0)
    plsc.subcore_barrier()
    buf0 = (sg0, sr0, dg0, ds0, el0, er0, wb0, None, rf0,
            (gf0, ge0, gr0, ssw0, ssr0))
    buf1 = (sg1, sr1, dg1, ds1, el1, er1, wb1, None, rf1,
            (gf1, ge1, gr1, ssw1, ssr1))
    _chunk_sweep(src_h, dst_h, tile_base, NCH2, 0, 0, N,
                 feat2_h, elr_sh, buf0, buf1, (src_c0, src_c1),
                 (dst_c0, dst_c1), ((cs0, cd0), (cs1, cd1)), s_sh, out_sh)
    plsc.subcore_barrier()
    pltpu.sync_copy(out_sh.at[pl.ds(s_id * chunk, chunk)],
                    u_h.at[pl.ds(c * NP + s_id * chunk, chunk)])
    pltpu.sync_copy(s_sh.at[pl.ds(s_id * chunk, chunk)],
                    s_out_h.at[pl.ds(c * NP + s_id * chunk, chunk)])


def _sc2(src, dst, elr2_i, feat2):
    dma = pltpu.SemaphoreType.DMA
    ik = jnp.int32
    f = jnp.float32
    kern = pl.kernel(
        _sc2_body,
        mesh=plsc.VectorSubcoreMesh(**_MESH),
        compiler_params=pltpu.CompilerParams(needs_layout_passes=False),
        out_type=[
            jax.ShapeDtypeStruct((2 * NP, 128), f),
            jax.ShapeDtypeStruct((2 * NP,), f),
        ],
        scratch_types=[
            pltpu.VMEM((CHB,), ik), pltpu.VMEM((CHB,), ik),
            pltpu.VMEM((CHB,), ik), pltpu.VMEM((CHB,), ik),
            pltpu.VMEM((KE,), ik), pltpu.VMEM((KE,), ik),
            pltpu.VMEM((KE,), ik), pltpu.VMEM((KE,), ik),
            pltpu.VMEM((KE,), ik), pltpu.VMEM((KE,), ik),
            pltpu.VMEM((KE,), f),
            pltpu.VMEM((KE, 128), f),
            pltpu.VMEM((KE,), ik), pltpu.VMEM((KE,), ik),
            pltpu.VMEM((KE,), ik), pltpu.VMEM((KE,), ik),
            pltpu.VMEM((KE,), ik), pltpu.VMEM((KE,), ik),
            pltpu.VMEM((KE,), f),
            pltpu.VMEM((KE, 128), f),
            pltpu.VMEM_SHARED((20480,), ik),
            pltpu.VMEM_SHARED((NP,), f),
            pltpu.VMEM_SHARED((NP, 128), f),
            dma, dma, dma, dma, dma, dma, dma, dma, dma, dma,
            dma, dma, dma, dma,
        ],
    )
    return kern(src, dst, elr2_i, feat2)


def kernel(x, edge_index, W1, aL1, aR1, W2, aL2, aR2):
    pad = EP - E
    pidx = jnp.arange(pad, dtype=jnp.int32)
    src = jnp.concatenate([edge_index[0], pidx * 37 % N])
    dst = jnp.concatenate([edge_index[1], N + pidx % (NP - N)])

    w1r = W1.reshape(D_IN, H1, HID).transpose(1, 0, 2)
    ul1 = jnp.einsum("hdk,hk->dh", w1r, aL1)
    ur1 = jnp.einsum("hdk,hk->dh", w1r, aR1)
    ulur = jnp.concatenate([ul1, ur1, jnp.zeros((D_IN, 112), jnp.float32)], 1)

    feat = _mm1a(x, w1r)
    eler = _mm1b(x, ulur)
    feat_flat = feat.reshape(H1 * N, 128)
    ep = jnp.pad(eler[:, :16], ((0, NPH - N), (0, 0)))
    order = [0, 8, 1, 9, 2, 10, 3, 11, 4, 12, 5, 13, 6, 14, 7, 15]
    elr = ep.T[jnp.array(order)].reshape(-1)
    elr_i = lax.bitcast_convert_type(elr, jnp.int32)

    u1 = _sc1(src, dst, elr_i, feat_flat)

    w2r = W2.reshape(H1, HID, OUT)
    ul2 = (W2 @ aL2[0]).reshape(H1, HID, 1)
    ur2 = (W2 @ aR2[0]).reshape(H1, HID, 1)
    w2e = jnp.concatenate(
        [w2r, ul2, ur2, jnp.zeros((H1, HID, 62), jnp.float32)], 2)

    m2 = _mm2(u1.reshape(H1, NP, 128), w2e)
    elr2 = jnp.concatenate(
        [m2[:, 64], m2[:, 65], jnp.zeros((480,), jnp.float32)])
    elr2_i = lax.bitcast_convert_type(elr2, jnp.int32)

    u2, s2 = _sc2(src, dst, elr2_i, m2)
    logits = _combine(u2.reshape(2, NP, 128),
                      s2.reshape(2, NP // 128, 128))
    return logits[:N]


# speedup vs baseline: 1.8109x; 1.0174x over previous
import jax
import jax.numpy as jnp
from jax import lax
from jax.experimental import pallas as pl
from jax.experimental.pallas import tpu as pltpu
from jax.experimental.pallas import tpu_sc as plsc

N = 10000
E = 320000
D_IN = 128
HID = 128
H1 = 8
OUT = 64
NEG = 0.2

NP = 10240
KE = 128
EP = 327680
EPT = EP // 16
CHB = 2560
NCH = EPT // CHB
BPC = CHB // KE
EPT2 = EP // 32
NCH2 = EPT2 // CHB
NPH = 10240

_MESH = dict(core_axis_name="c", subcore_axis_name="s")


def _mm1a_body(x_ref, w_ref, o_ref):
    o_ref[0] = jnp.dot(x_ref[...], w_ref[0],
                       preferred_element_type=jnp.float32,
                       precision=lax.Precision.HIGHEST)


def _mm1a(x, w1r):
    bn = 400
    return pl.pallas_call(
        _mm1a_body,
        grid=(H1, N // bn),
        in_specs=[
            pl.BlockSpec((bn, D_IN), lambda h, i: (i, 0)),
            pl.BlockSpec((1, D_IN, 128), lambda h, i: (h, 0, 0)),
        ],
        out_specs=pl.BlockSpec((1, bn, 128), lambda h, i: (h, i, 0)),
        out_shape=jax.ShapeDtypeStruct((H1, N, 128), jnp.float32),
    )(x, w1r)


def _mm1b_body(x_ref, w_ref, o_ref):
    o_ref[...] = jnp.dot(x_ref[...], w_ref[...],
                         preferred_element_type=jnp.float32,
                         precision=lax.Precision.HIGHEST)


def _mm1b(x, ulur):
    bn = 400
    return pl.pallas_call(
        _mm1b_body,
        grid=(N // bn,),
        in_specs=[
            pl.BlockSpec((bn, D_IN), lambda i: (i, 0)),
            pl.BlockSpec((D_IN, 128), lambda i: (0, 0)),
        ],
        out_specs=pl.BlockSpec((bn, 128), lambda i: (i, 0)),
        out_shape=jax.ShapeDtypeStruct((N, 128), jnp.float32),
    )(x, ulur)


def _mm2_body(h_ref, s_ref, w_ref, o_ref):
    bn = o_ref.shape[0]
    acc = jnp.zeros((bn, 128), jnp.float32)
    for hh in range(H1):
        sh = s_ref[hh].reshape(bn) + 1e-9
        a = h_ref[hh] / sh[:, None]
        a = jnp.where(a > 0, a, jnp.exp(a) - 1.0)
        acc = acc + jnp.dot(a, w_ref[hh],
                            preferred_element_type=jnp.float32,
                            precision=lax.Precision.HIGHEST)
    o_ref[...] = acc


def _mm2(h1, s1, w2e):
    bn = 1024
    return pl.pallas_call(
        _mm2_body,
        grid=(NP // bn,),
        in_specs=[
            pl.BlockSpec((H1, bn, 128), lambda i: (0, i, 0)),
            pl.BlockSpec((H1, bn // 128, 128), lambda i: (0, i, 0)),
            pl.BlockSpec((H1, 128, 128), lambda i: (0, 0, 0)),
        ],
        out_specs=pl.BlockSpec((bn, 128), lambda i: (i, 0)),
        out_shape=jax.ShapeDtypeStruct((NP, 128), jnp.float32),
    )(h1, s1, w2e)


def _comb_body(u_ref, s_ref, o_ref):
    bn = u_ref.shape[1]
    su = s_ref[0].reshape(bn) + s_ref[1].reshape(bn) + 1e-9
    o_ref[...] = (u_ref[0, :, :OUT] + u_ref[1, :, :OUT]) / su[:, None]


def _combine(u, s):
    bn = 1024
    return pl.pallas_call(
        _comb_body,
        grid=(NP // bn,),
        in_specs=[
            pl.BlockSpec((2, bn, 128), lambda i: (0, i, 0)),
            pl.BlockSpec((2, bn // 128, 128), lambda i: (0, i, 0)),
        ],
        out_specs=pl.BlockSpec((bn, OUT), lambda i: (i, 0)),
        out_shape=jax.ShapeDtypeStruct((NP, OUT), jnp.float32),
    )(u, s)


def _gather_dst(buf):
    return buf[7] if buf[7] is not None else buf[8]


def _stage_a(b, feat_off, el_off, er_off, src_ch, dst_ch, feat_h, elr_sh,
             buf, wait_scatter, s_sh, out_sh):
    sg, sr, dg, ds, elv, erv, wb, rbf, rf, sems = buf
    if wait_scatter is True:
        pltpu.make_async_copy(wb, s_sh.at[ds], sems[3]).wait()
        pltpu.make_async_copy(rf, out_sh.at[ds], sems[4]).wait()
    elif wait_scatter is not None:
        @pl.when(wait_scatter)
        def _():
            pltpu.make_async_copy(wb, s_sh.at[ds], sems[3]).wait()
            pltpu.make_async_copy(rf, out_sh.at[ds], sems[4]).wait()
    for v in range(KE // 16):
        sl = pl.ds(v * 16, 16)
        s16 = src_ch[pl.ds(b * KE + v * 16, 16)]
        d16 = dst_ch[pl.ds(b * KE + v * 16, 16)]
        sg[sl] = s16 + feat_off
        sr[sl] = s16 + el_off
        dg[sl] = d16 + er_off
        ds[sl] = d16
    pltpu.async_copy(feat_h.at[sg], _gather_dst(buf), sems[0])
    pltpu.async_copy(elr_sh.at[sr], elv, sems[1])
    pltpu.async_copy(elr_sh.at[dg], erv, sems[2])


def _stage_b(feat_h, elr_sh, buf, s_sh, out_sh):
    sg, sr, dg, ds, elv, erv, wb, rbf, rf, sems = buf
    pltpu.make_async_copy(elr_sh.at[sr], elv, sems[1]).wait()
    pltpu.make_async_copy(elr_sh.at[dg], erv, sems[2]).wait()
    for v in range(KE // 16):
        sl = pl.ds(v * 16, 16)
        e16 = (plsc.bitcast(elv[sl], jnp.float32)
               + plsc.bitcast(erv[sl], jnp.float32))
        wb[sl] = jnp.exp(jnp.maximum(e16, NEG * e16))
    pltpu.make_async_copy(feat_h.at[sg], _gather_dst(buf), sems[0]).wait()

    if rbf is not None:
        def scale_row(r, _):
            a = plsc.load_gather(wb, [jnp.full((16,), r, jnp.int32)])
            for q in range(4):
                ab = rbf[r, pl.ds(q * 32, 32)]
                lo, hi = plsc.unpack(ab, format=plsc.PackFormat.INTERLEAVED)
                rf[r, pl.ds(q * 32, 16)] = lo * a
                rf[r, pl.ds(q * 32 + 16, 16)] = hi * a
            return 0
    else:
        def scale_row(r, _):
            a = plsc.load_gather(wb, [jnp.full((16,), r, jnp.int32)])
            for v in range(8):
                sl = pl.ds(v * 16, 16)
                rf[r, sl] = rf[r, sl] * a
            return 0

    lax.fori_loop(0, KE, scale_row, 0)
    pltpu.async_copy(wb, s_sh.at[ds], sems[3], add=True)
    pltpu.async_copy(rf, out_sh.at[ds], sems[4], add=True)


def _drain_scatters(bufs, s_sh, out_sh):
    for buf in bufs:
        _, _, _, ds, _, _, wb, _, rf, sems = buf
        pltpu.make_async_copy(wb, s_sh.at[ds], sems[3]).wait()
        pltpu.make_async_copy(rf, out_sh.at[ds], sems[4]).wait()


def _zero_rows_buf(rw):
    def st(g, _):
        for v in range(8):
            rw[g, pl.ds(v * 16, 16)] = jnp.zeros((16,), jnp.float32)
        return 0
    lax.fori_loop(0, KE, st, 0)


def _zero_accumulators(s_id, zb, rw, s_sh, out_sh):
    for k in range(640 // KE):
        pltpu.sync_copy(zb, s_sh.at[pl.ds(s_id * 640 + k * KE, KE)])
        pltpu.sync_copy(rw, out_sh.at[pl.ds(s_id * 640 + k * KE, KE)])


def _chunk_sweep(src_h, dst_h, tile_base, nch, feat_off, el_off, er_off,
                 feat_h, elr_sh, buf0, buf1, src_ch, dst_ch, csems,
                 s_sh, out_sh):
    for ci in range(nch):
        par = ci % 2
        sc, dc = src_ch[par], dst_ch[par]
        pltpu.make_async_copy(src_h.at[pl.ds(tile_base, CHB)], sc,
                              csems[par][0]).wait()
        pltpu.make_async_copy(dst_h.at[pl.ds(tile_base, CHB)], dc,
                              csems[par][1]).wait()
        if ci + 1 < nch:
            nb = (ci + 1) % 2
            off = tile_base + (ci + 1) * CHB
            pltpu.async_copy(src_h.at[pl.ds(off, CHB)],
                             src_ch[nb], csems[nb][0])
            pltpu.async_copy(dst_h.at[pl.ds(off, CHB)],
                             dst_ch[nb], csems[nb][1])
        _stage_a(0, feat_off, el_off, er_off, sc, dc, feat_h, elr_sh, buf0,
                 None if ci == 0 else True, s_sh, out_sh)

        def pair(p, _):
            b0 = 2 * p
            _stage_a(b0 + 1, feat_off, el_off, er_off, sc, dc, feat_h,
                     elr_sh, buf1, (p > 0) if ci == 0 else True,
                     s_sh, out_sh)
            _stage_b(feat_h, elr_sh, buf0, s_sh, out_sh)

            @pl.when(p < BPC // 2 - 1)
            def _():
                _stage_a(b0 + 2, feat_off, el_off, er_off, sc, dc, feat_h,
                         elr_sh, buf0, True, s_sh, out_sh)
            _stage_b(feat_h, elr_sh, buf1, s_sh, out_sh)
            return 0

        lax.fori_loop(0, BPC // 2, pair, 0)
    _drain_scatters((buf0, buf1), s_sh, out_sh)


def _sc1_body(src_h, dst_h, elr_h, feat_h, out_h, s1_h,
              src_c0, src_c1, dst_c0, dst_c1,
              sg0, sr0, dg0, ds0, el0, er0, wb0, rf0,
              sg1, sr1, dg1, ds1, el1, er1, wb1, rf1,
              elr_sh, s_sh, out_sh,
              gf0, ge0, gr0, ssw0, ssr0, gf1, ge1, gr1, ssw1, ssr1,
              cs0, cd0, cs1, cd1):
    c = lax.axis_index("c")
    s_id = lax.axis_index("s")
    buf0 = (sg0, sr0, dg0, ds0, el0, er0, wb0, None, rf0,
            (gf0, ge0, gr0, ssw0, ssr0))
    buf1 = (sg1, sr1, dg1, ds1, el1, er1, wb1, None, rf1,
            (gf1, ge1, gr1, ssw1, ssr1))
    src_ch = (src_c0, src_c1)
    dst_ch = (dst_c0, dst_c1)
    csems = ((cs0, cd0), (cs1, cd1))
    tile_base = s_id * EPT

    def head(hh, _):
        h = c * 4 + hh
        for tb in (0, NPH):
            g = h * 2 * NPH + tb + s_id * 640
            pltpu.sync_copy(elr_h.at[pl.ds(g, 640)], src_c0.at[pl.ds(0, 640)])
            pltpu.sync_copy(src_c0.at[pl.ds(0, 640)],
                            elr_sh.at[pl.ds(tb + s_id * 640, 640)])
        _zero_rows_buf(rf0)
        for v in range(KE // 16):
            wb0[pl.ds(v * 16, 16)] = jnp.zeros((16,), jnp.float32)
        _zero_accumulators(s_id, wb0, rf0, s_sh, out_sh)
        pltpu.async_copy(src_h.at[pl.ds(tile_base, CHB)], src_c0, cs0)
        pltpu.async_copy(dst_h.at[pl.ds(tile_base, CHB)], dst_c0, cd0)
        plsc.subcore_barrier()
        _chunk_sweep(src_h, dst_h, tile_base, NCH, h * N, 0, NPH,
                     feat_h, elr_sh, buf0, buf1, src_ch, dst_ch, csems,
                     s_sh, out_sh)
        plsc.subcore_barrier()
        ofs = s_id * 640
        pltpu.sync_copy(out_sh.at[pl.ds(ofs, 640)],
                        out_h.at[pl.ds(h * NP + ofs, 640)])
        pltpu.sync_copy(s_sh.at[pl.ds(ofs, 640)],
                        s1_h.at[pl.ds(h * NP + ofs, 640)])
        plsc.subcore_barrier()
        return 0

    lax.fori_loop(0, 4, head, 0)


def _sc1(src, dst, elr_i, feat):
    dma = pltpu.SemaphoreType.DMA
    ik = jnp.int32
    f = jnp.float32
    kern = pl.kernel(
        _sc1_body,
        mesh=plsc.VectorSubcoreMesh(**_MESH),
        compiler_params=pltpu.CompilerParams(needs_layout_passes=False),
        out_type=[
            jax.ShapeDtypeStruct((H1 * NP, 128), f),
            jax.ShapeDtypeStruct((H1 * NP,), f),
        ],
        scratch_types=[
            pltpu.VMEM((CHB,), ik), pltpu.VMEM((CHB,), ik),
            pltpu.VMEM((CHB,), ik), pltpu.VMEM((CHB,), ik),
            pltpu.VMEM((KE,), ik), pltpu.VMEM((KE,), ik),
            pltpu.VMEM((KE,), ik), pltpu.VMEM((KE,), ik),
            pltpu.VMEM((KE,), ik), pltpu.VMEM((KE,), ik),
            pltpu.VMEM((KE,), f),
            pltpu.VMEM((KE, 128), f),
            pltpu.VMEM((KE,), ik), pltpu.VMEM((KE,), ik),
            pltpu.VMEM((KE,), ik), pltpu.VMEM((KE,), ik),
            pltpu.VMEM((KE,), ik), pltpu.VMEM((KE,), ik),
            pltpu.VMEM((KE,), f),
            pltpu.VMEM((KE, 128), f),
            pltpu.VMEM_SHARED((2 * NPH,), ik),
            pltpu.VMEM_SHARED((NP,), f),
            pltpu.VMEM_SHARED((NP, 128), f),
            dma, dma, dma, dma, dma, dma, dma, dma, dma, dma,
            dma, dma, dma, dma,
        ],
    )
    return kern(src, dst, elr_i, feat)


def _sc2_body(src_h, dst_h, elr2_h, feat2_h, u_h, s_out_h,
              src_c0, src_c1, dst_c0, dst_c1,
              sg0, sr0, dg0, ds0, el0, er0, wb0, rf0,
              sg1, sr1, dg1, ds1, el1, er1, wb1, rf1,
              elr_sh, s_sh, out_sh,
              gf0, ge0, gr0, ssw0, ssr0, gf1, ge1, gr1, ssw1, ssr1,
              cs0, cd0, cs1, cd1):
    c = lax.axis_index("c")
    s_id = lax.axis_index("s")
    chunk = NP // 16
    pltpu.sync_copy(elr2_h.at[pl.ds(s_id * 1280, 1280)],
                    src_c0.at[pl.ds(0, 1280)])
    pltpu.sync_copy(src_c0.at[pl.ds(0, 1280)],
                    elr_sh.at[pl.ds(s_id * 1280, 1280)])
    _zero_rows_buf(rf0)
    for v in range(KE // 16):
        wb0[pl.ds(v * 16, 16)] = jnp.zeros((16,), jnp.float32)
    _zero_accumulators(s_id, wb0, rf0, s_sh, out_sh)
    wid = c * 16 + s_id
    tile_base = wid * EPT2
    pltpu.async_copy(src_h.at[pl.ds(tile_base, CHB)], src_c0, cs0)
    pltpu.async_copy(dst_h.at[pl.ds(tile_base, CHB)], dst_c0, cd0)
    plsc.subcore_barrier()
    buf0 = (sg0, sr0, dg0, ds0, el0, er0, wb0, None, rf0,
            (gf0, ge0, gr0, ssw0, ssr0))
    buf1 = (sg1, sr1, dg1, ds1, el1, er1, wb1, None, rf1,
            (gf1, ge1, gr1, ssw1, ssr1))
    _chunk_sweep(src_h, dst_h, tile_base, NCH2, 0, 0, N,
                 feat2_h, elr_sh, buf0, buf1, (src_c0, src_c1),
                 (dst_c0, dst_c1), ((cs0, cd0), (cs1, cd1)), s_sh, out_sh)
    plsc.subcore_barrier()
    pltpu.sync_copy(out_sh.at[pl.ds(s_id * chunk, chunk)],
                    u_h.at[pl.ds(c * NP + s_id * chunk, chunk)])
    pltpu.sync_copy(s_sh.at[pl.ds(s_id * chunk, chunk)],
                    s_out_h.at[pl.ds(c * NP + s_id * chunk, chunk)])


def _sc2(src, dst, elr2_i, feat2):
    dma = pltpu.SemaphoreType.DMA
    ik = jnp.int32
    f = jnp.float32
    kern = pl.kernel(
        _sc2_body,
        mesh=plsc.VectorSubcoreMesh(**_MESH),
        compiler_params=pltpu.CompilerParams(needs_layout_passes=False),
        out_type=[
            jax.ShapeDtypeStruct((2 * NP, 128), f),
            jax.ShapeDtypeStruct((2 * NP,), f),
        ],
        scratch_types=[
            pltpu.VMEM((CHB,), ik), pltpu.VMEM((CHB,), ik),
            pltpu.VMEM((CHB,), ik), pltpu.VMEM((CHB,), ik),
            pltpu.VMEM((KE,), ik), pltpu.VMEM((KE,), ik),
            pltpu.VMEM((KE,), ik), pltpu.VMEM((KE,), ik),
            pltpu.VMEM((KE,), ik), pltpu.VMEM((KE,), ik),
            pltpu.VMEM((KE,), f),
            pltpu.VMEM((KE, 128), f),
            pltpu.VMEM((KE,), ik), pltpu.VMEM((KE,), ik),
            pltpu.VMEM((KE,), ik), pltpu.VMEM((KE,), ik),
            pltpu.VMEM((KE,), ik), pltpu.VMEM((KE,), ik),
            pltpu.VMEM((KE,), f),
            pltpu.VMEM((KE, 128), f),
            pltpu.VMEM_SHARED((20480,), ik),
            pltpu.VMEM_SHARED((NP,), f),
            pltpu.VMEM_SHARED((NP, 128), f),
            dma, dma, dma, dma, dma, dma, dma, dma, dma, dma,
            dma, dma, dma, dma,
        ],
    )
    return kern(src, dst, elr2_i, feat2)


def kernel(x, edge_index, W1, aL1, aR1, W2, aL2, aR2):
    pad = EP - E
    pidx = jnp.arange(pad, dtype=jnp.int32)
    src = jnp.concatenate([edge_index[0], pidx * 37 % N])
    dst = jnp.concatenate([edge_index[1], N + pidx % (NP - N)])

    w1r = W1.reshape(D_IN, H1, HID).transpose(1, 0, 2)
    ul1 = jnp.einsum("hdk,hk->dh", w1r, aL1)
    ur1 = jnp.einsum("hdk,hk->dh", w1r, aR1)
    ulur = jnp.concatenate([ul1, ur1, jnp.zeros((D_IN, 112), jnp.float32)], 1)

    feat = _mm1a(x, w1r)
    eler = _mm1b(x, ulur)
    feat_flat = feat.reshape(H1 * N, 128)
    ep = jnp.pad(eler[:, :16], ((0, NPH - N), (0, 0)))
    order = [0, 8, 1, 9, 2, 10, 3, 11, 4, 12, 5, 13, 6, 14, 7, 15]
    elr = ep.T[jnp.array(order)].reshape(-1)
    elr_i = lax.bitcast_convert_type(elr, jnp.int32)

    u1, s1 = _sc1(src, dst, elr_i, feat_flat)

    w2r = W2.reshape(H1, HID, OUT)
    ul2 = (W2 @ aL2[0]).reshape(H1, HID, 1)
    ur2 = (W2 @ aR2[0]).reshape(H1, HID, 1)
    w2e = jnp.concatenate(
        [w2r, ul2, ur2, jnp.zeros((H1, HID, 62), jnp.float32)], 2)

    m2 = _mm2(u1.reshape(H1, NP, 128),
              s1.reshape(H1, NP // 128, 128), w2e)
    elr2 = jnp.concatenate(
        [m2[:N, 64], m2[:N, 65], jnp.zeros((480,), jnp.float32)])
    elr2_i = lax.bitcast_convert_type(elr2, jnp.int32)

    u2, s2 = _sc2(src, dst, elr2_i, m2)
    logits = _combine(u2.reshape(2, NP, 128),
                      s2.reshape(2, NP // 128, 128))
    return logits[:N]
